# baseline (speedup 1.0000x reference)
import functools

import jax
import jax.numpy as jnp
from jax import lax
from jax.experimental import pallas as pl
from jax.experimental.pallas import tpu as pltpu
from jax.experimental.pallas import tpu_sc as plsc

B = 16384
D = 16
NC = 2
NS = 16
NW = NC * NS
BPW = B // NW
NCH = 4
CH = BPW // NCH


def _mf_body(gid_hbm, jid_hbm, g1_hbm, j1_hbm, g2_hbm, j2_hbm,
             gb1_hbm, jb1_hbm, gb2_hbm, jb2_hbm, w_hbm,
             out_hbm,
             gidx, jidx, g1v, j1v, g2v, j2v,
             gb1v, jb1v, gb2v, jb2v, wv, outv, sem):
    wid = lax.axis_index("s") * NC + lax.axis_index("c")
    base = wid * BPW

    descs = [pltpu.async_copy(w_hbm, wv, sem)]
    for c in range(NCH):
        descs.append(pltpu.async_copy(
            gid_hbm.at[pl.ds(base + c * CH, CH)], gidx.at[c], sem))
        descs.append(pltpu.async_copy(
            jid_hbm.at[pl.ds(base + c * CH, CH)], jidx.at[c], sem))
    for d_ in descs:
        d_.wait()

    descs = []
    for c in range(NCH):
        gi = gidx.at[c]
        ji = jidx.at[c]
        sl = pl.ds(c * CH, CH)
        descs.append(pltpu.async_copy(g1_hbm.at[gi], g1v.at[sl], sem))
        descs.append(pltpu.async_copy(j1_hbm.at[ji], j1v.at[sl], sem))
        descs.append(pltpu.async_copy(g2_hbm.at[gi], g2v.at[sl], sem))
        descs.append(pltpu.async_copy(j2_hbm.at[ji], j2v.at[sl], sem))
        descs.append(pltpu.async_copy(gb1_hbm.at[gi], gb1v.at[sl], sem))
        descs.append(pltpu.async_copy(jb1_hbm.at[ji], jb1v.at[sl], sem))
        descs.append(pltpu.async_copy(gb2_hbm.at[gi], gb2v.at[sl], sem))
        descs.append(pltpu.async_copy(jb2_hbm.at[ji], jb2v.at[sl], sem))
    for d_ in descs:
        d_.wait()

    iota16 = lax.iota(jnp.int32, 16)
    zeros16 = jnp.zeros((16,), jnp.int32)
    cv = wv[2]
    wa_vec = wv[0]
    wb_vec = wv[1]
    was = [wa_vec[d] for d in range(D)]
    wbs = [wb_vec[d] for d in range(D)]

    def blk_body(blk, carry):
        pv = iota16 + blk * 16
        bsl = pl.ds(blk * 16, 16)
        acc = cv + gb1v[bsl] + gb2v[bsl] + jb1v[bsl] + jb2v[bsl]
        for d in range(D):
            dsplat = jnp.full((16,), d, jnp.int32)
            a = plsc.load_gather(g1v, [pv, dsplat])
            b = plsc.load_gather(j1v, [pv, dsplat])
            c2 = plsc.load_gather(g2v, [pv, dsplat])
            e = plsc.load_gather(j2v, [pv, dsplat])
            acc = acc + a * b * was[d] + c2 * e * wbs[d]
        outv[pl.ds(blk * 16, 16)] = acc
        return carry

    lax.fori_loop(0, BPW // 16, blk_body, 0)

    pltpu.sync_copy(outv, out_hbm.at[pl.ds(base, BPW)])


_mf_call = functools.partial(
    pl.kernel,
    out_type=jax.ShapeDtypeStruct((B,), jnp.float32),
    mesh=plsc.VectorSubcoreMesh(core_axis_name="c", subcore_axis_name="s",
                                num_cores=NC, num_subcores=NS),
    scratch_types=[
        pltpu.VMEM((NCH, CH), jnp.int32),
        pltpu.VMEM((NCH, CH), jnp.int32),
        pltpu.VMEM((BPW, D), jnp.float32),
        pltpu.VMEM((BPW, D), jnp.float32),
        pltpu.VMEM((BPW, D), jnp.float32),
        pltpu.VMEM((BPW, D), jnp.float32),
        pltpu.VMEM((BPW,), jnp.float32),
        pltpu.VMEM((BPW,), jnp.float32),
        pltpu.VMEM((BPW,), jnp.float32),
        pltpu.VMEM((BPW,), jnp.float32),
        pltpu.VMEM((3, D), jnp.float32),
        pltpu.VMEM((BPW,), jnp.float32),
        pltpu.SemaphoreType.DMA,
    ],
    compiler_params=pltpu.CompilerParams(needs_layout_passes=False,
                                         use_tc_tiling_on_sc=False),
)(_mf_body)


def kernel(geek_id, job_id, geek_emb1, job_emb1, geek_emb2, job_emb2,
           geek_b1, job_b1, geek_b2, job_b2, W1, b1, W2, b2, miu1, miu2):
    w = (W1 @ W2)[:, 0]
    const = (b1 @ W2)[0] + b2[0] + miu1 + miu2
    wpack = jnp.stack([w[:D], w[D:], jnp.full((D,), const, jnp.float32)])
    return _mf_call(geek_id.astype(jnp.int32), job_id.astype(jnp.int32),
                    geek_emb1, job_emb1, geek_emb2, job_emb2,
                    geek_b1[:, 0], job_b1[:, 0], geek_b2[:, 0], job_b2[:, 0],
                    wpack)
